# Initial kernel scaffold; baseline (speedup 1.0000x reference)
import jax
import jax.numpy as jnp
from jax.experimental import pallas as pl


def kernel(atom_list, bond_list, atom_degree_list, bond_degree_list, atom_mask, descriptors, params):
    raise NotImplementedError("write your pallas kernel here")


# speedup vs baseline: 8.4861x; 8.4861x over previous
import jax
import jax.numpy as jnp
from jax.experimental import pallas as pl

B = 256
L = 128
M = 6
D = 200
IN_FEAT = 39
BOND_FEAT = 10
RADIUS = 3
T_STEPS = 2
NB = 8
NEG = -900000000.0


def _dot(a, b):
    return jax.lax.dot_general(
        a, b, (((1,), (0,)), ((), ())),
        preferred_element_type=jnp.float32,
        precision=jax.lax.Precision.DEFAULT)


def _dot_exact(a, b):
    return jax.lax.dot_general(
        a, b, (((1,), (0,)), ((), ())),
        preferred_element_type=jnp.float32,
        precision=jax.lax.Precision.HIGHEST)


def _leaky(x):
    return jnp.where(x >= 0, x, 0.01 * x)


def _elu(x):
    return jnp.where(x > 0, x, jnp.exp(jnp.minimum(x, 0.0)) - 1.0)


def _gnn_body(atom_ref, bond_ref, aidx_ref, bidx_ref, mask_ref,
              aW, aB, nWa, nWb, nB_,
              g_ih, g_hh, g_bih, g_bhh,
              al_wA, al_wN_col, al_wN_row, al_b,
              at_W, at_b,
              mg_ih, mg_hh, mg_bih, mg_bhh,
              ma_wA, ma_wN, ma_b,
              mt_W, mt_b,
              out_af, out_mf):
    f32 = jnp.float32

    def gru(x, h, ih, hh, bih, bhh, base):
        r = jax.nn.sigmoid(_dot(x, ih[base + 0]) + bih[base + 0]
                           + _dot(h, hh[base + 0]) + bhh[base + 0])
        z = jax.nn.sigmoid(_dot(x, ih[base + 1]) + bih[base + 1]
                           + _dot(h, hh[base + 1]) + bhh[base + 1])
        n = jnp.tanh(_dot(x, ih[base + 2]) + bih[base + 2]
                     + r * (_dot(h, hh[base + 2]) + bhh[base + 2]))
        return (1.0 - z) * n + z * h

    x2 = atom_ref[...].reshape(NB * L, IN_FEAT)
    bd2 = bond_ref[...].reshape(NB * L, BOND_FEAT)
    af = _leaky(_dot(x2, aW[...]) + aB[...])
    pa = _dot(x2, nWa[...])
    pb = _dot(bd2, nWb[...]) + nB_[...]

    iota = jax.lax.broadcasted_iota(jnp.int32, (L, L), 1)

    sA0 = _dot(af, al_wA[0]) + al_b[0]
    ctx_rows = []
    for mol in range(NB):
        sl = slice(mol * L, (mol + 1) * L)
        pa_m, pb_m = pa[sl, :], pb[sl, :]
        aidx_m = aidx_ref[mol]
        bidx_m = bidx_ref[mol]
        sA_m = sA0[sl, :]
        nfs, scores, pads = [], [], []
        for m in range(M):
            a_col = aidx_m[:, m:m + 1]
            b_col = bidx_m[:, m:m + 1]
            oh_a = (a_col == iota).astype(f32)
            oh_b = (b_col == iota).astype(f32)
            nf = _leaky(_dot_exact(oh_a, pa_m) + _dot_exact(oh_b, pb_m))
            s_nf = _dot(nf, al_wN_col[0])
            pad = a_col == (L - 1)
            score = _leaky(sA_m + s_nf) + jnp.where(pad, NEG, 0.0)
            nfs.append(nf)
            scores.append(score)
            pads.append(pad)
        mx = scores[0]
        for m in range(1, M):
            mx = jnp.maximum(mx, scores[m])
        es = [jnp.exp(s - mx) for s in scores]
        ssum = es[0]
        for m in range(1, M):
            ssum = ssum + es[m]
        ctx = jnp.zeros((L, D), f32)
        for m in range(M):
            w = jnp.where(pads[m], 0.0, es[m] / ssum)
            nbr_t = _dot(nfs[m], at_W[0]) + at_b[0]
            ctx = ctx + w * nbr_t
        ctx_rows.append(ctx)
    ctx_all = _elu(jnp.concatenate(ctx_rows, axis=0))
    h = gru(ctx_all, af, g_ih, g_hh, g_bih, g_bhh, 0)
    act = jax.nn.relu(h)

    for r in range(1, RADIUS):
        proj = _dot(act, at_W[r])
        sA = _dot(act, al_wA[r]) + al_b[r]
        sN = _dot(act, al_wN_col[r])
        ctx_rows = []
        for mol in range(NB):
            sl = slice(mol * L, (mol + 1) * L)
            proj_m = proj[sl, :]
            sA_m, sN_m = sA[sl, :], sN[sl, :]
            aidx_m = aidx_ref[mol]
            ohs, scores, pads = [], [], []
            for m in range(M):
                a_col = aidx_m[:, m:m + 1]
                oh = (a_col == iota).astype(f32)
                s_g = _dot_exact(oh, sN_m)
                pad = a_col == (L - 1)
                score = _leaky(sA_m + s_g) + jnp.where(pad, NEG, 0.0)
                ohs.append(oh)
                scores.append(score)
                pads.append(pad)
            mx = scores[0]
            for m in range(1, M):
                mx = jnp.maximum(mx, scores[m])
            es = [jnp.exp(s - mx) for s in scores]
            ssum = es[0]
            for m in range(1, M):
                ssum = ssum + es[m]
            ctx = jnp.zeros((L, D), f32)
            for m in range(M):
                w = jnp.where(pads[m], 0.0, es[m] / ssum)
                nbr_t = _dot_exact(ohs[m], proj_m) + at_b[r]
                ctx = ctx + w * nbr_t
            ctx_rows.append(ctx)
        ctx_all = _elu(jnp.concatenate(ctx_rows, axis=0))
        h = gru(ctx_all, h, g_ih, g_hh, g_bih, g_bhh, 3 * r)
        act = jax.nn.relu(h)

    mf_rows = []
    for mol in range(NB):
        sl = slice(mol * L, (mol + 1) * L)
        mask_m = mask_ref[mol]
        mf_rows.append(jnp.sum(act[sl, :] * mask_m, axis=0, keepdims=True))
    mf = jnp.concatenate(mf_rows, axis=0)
    act_t = _dot(act, mt_W[...]) + mt_b[...]
    sN_mol = _dot(act, ma_wN[...])
    am = jax.nn.relu(mf)
    for _t in range(T_STEPS):
        sA_t = _dot(am, ma_wA[...]) + ma_b[...]
        mc_rows = []
        for mol in range(NB):
            sl = slice(mol * L, (mol + 1) * L)
            mask_m = mask_ref[mol]
            score = _leaky(sA_t[mol:mol + 1, :] + sN_mol[sl, :]) \
                + jnp.where(mask_m == 0.0, NEG, 0.0)
            mx = jnp.max(score, axis=0, keepdims=True)
            e = jnp.exp(score - mx)
            w = e / jnp.sum(e, axis=0, keepdims=True) * mask_m
            mc_rows.append(jnp.sum(w * act_t[sl, :], axis=0, keepdims=True))
        mc = _elu(jnp.concatenate(mc_rows, axis=0))
        mf = gru(mc, mf, mg_ih, mg_hh, mg_bih, mg_bhh, 0)
        am = jax.nn.relu(mf)

    out_af[...] = h.reshape(NB, L, D)
    out_mf[...] = mf


def _heads_body(mf_ref, desc_ref,
                g1W, g1b, s1W, s1b, s2W, s2b, s3W, s3b,
                f1Wa, f1Wb, f1b, f2W, f2b, oW, ob,
                out_ref):
    x = _dot(mf_ref[...], g1W[...]) + g1b[...]
    d1 = jax.nn.relu(_dot(desc_ref[...], s1W[...]) + s1b[...])
    d2 = jax.nn.relu(_dot(d1, s2W[...]) + s2b[...])
    d3 = _dot(d2, s3W[...]) + s3b[...]
    m1 = jax.nn.relu(_dot(x, f1Wa[...]) + _dot(d3, f1Wb[...]) + f1b[...])
    m2 = jax.nn.relu(_dot(m1, f2W[...]) + f2b[...])
    out_ref[...] = _dot(m2, oW[...]) + ob[...]


def kernel(atom_list, bond_list, atom_degree_list, bond_degree_list,
           atom_mask, descriptors, params):
    p = params
    f32 = jnp.float32
    aidx = atom_degree_list.astype(jnp.int32)
    bidx = bond_degree_list.astype(jnp.int32)
    mask3 = atom_mask.astype(f32).reshape(B, L, 1)

    def row(b):
        return b.reshape(1, -1).astype(f32)

    def pack_gru(gs):
        ih = jnp.stack([g["w_ih"][i * D:(i + 1) * D, :].T
                        for g in gs for i in range(3)])
        hh = jnp.stack([g["w_hh"][i * D:(i + 1) * D, :].T
                        for g in gs for i in range(3)])
        bih = jnp.stack([g["b_ih"][i * D:(i + 1) * D].reshape(1, D)
                         for g in gs for i in range(3)])
        bhh = jnp.stack([g["b_hh"][i * D:(i + 1) * D].reshape(1, D)
                         for g in gs for i in range(3)])
        return ih, hh, bih, bhh

    g_ih, g_hh, g_bih, g_bhh = pack_gru(p["gru"])
    mg_ih, mg_hh, mg_bih, mg_bhh = pack_gru([p["mol_gru"]])

    al_wA = jnp.stack([a["W"][:D, :] for a in p["align"]])
    al_wN_col = jnp.stack([a["W"][D:, :] for a in p["align"]])
    al_wN_row = jnp.stack([a["W"][D:, :].T for a in p["align"]])
    al_b = jnp.stack([a["b"].reshape(1, 1) for a in p["align"]])
    at_W = jnp.stack([a["W"] for a in p["attend"]])
    at_b = jnp.stack([a["b"].reshape(1, D) for a in p["attend"]])

    nWa = p["nbr_lin"]["W"][:IN_FEAT, :]
    nWb = p["nbr_lin"]["W"][IN_FEAT:, :]

    grid = B // NB
    const2 = lambda shape: pl.BlockSpec(shape, lambda i: (0,) * len(shape))
    gnn_in_specs = [
        pl.BlockSpec((NB, L, IN_FEAT), lambda i: (i, 0, 0)),
        pl.BlockSpec((NB, L, BOND_FEAT), lambda i: (i, 0, 0)),
        pl.BlockSpec((NB, L, M), lambda i: (i, 0, 0)),
        pl.BlockSpec((NB, L, M), lambda i: (i, 0, 0)),
        pl.BlockSpec((NB, L, 1), lambda i: (i, 0, 0)),
        const2((IN_FEAT, D)), const2((1, D)),
        const2((IN_FEAT, D)), const2((BOND_FEAT, D)), const2((1, D)),
        const2((9, D, D)), const2((9, D, D)),
        const2((9, 1, D)), const2((9, 1, D)),
        const2((3, D, 1)), const2((3, D, 1)), const2((3, 1, D)),
        const2((3, 1, 1)),
        const2((3, D, D)), const2((3, 1, D)),
        const2((3, D, D)), const2((3, D, D)),
        const2((3, 1, D)), const2((3, 1, D)),
        const2((D, 1)), const2((D, 1)), const2((1, 1)),
        const2((D, D)), const2((1, D)),
    ]
    atom_feature, mol_feature = pl.pallas_call(
        _gnn_body,
        grid=(grid,),
        in_specs=gnn_in_specs,
        out_specs=[
            pl.BlockSpec((NB, L, D), lambda i: (i, 0, 0)),
            pl.BlockSpec((NB, D), lambda i: (i, 0)),
        ],
        out_shape=[
            jax.ShapeDtypeStruct((B, L, D), f32),
            jax.ShapeDtypeStruct((B, D), f32),
        ],
    )(atom_list.astype(f32), bond_list.astype(f32), aidx, bidx, mask3,
      p["atom_lin"]["W"], row(p["atom_lin"]["b"]),
      nWa, nWb, row(p["nbr_lin"]["b"]),
      g_ih, g_hh, g_bih, g_bhh,
      al_wA, al_wN_col, al_wN_row, al_b,
      at_W, at_b,
      mg_ih, mg_hh, mg_bih, mg_bhh,
      p["mol_align"]["W"][:D, :], p["mol_align"]["W"][D:, :],
      p["mol_align"]["b"].reshape(1, 1),
      p["mol_attend"]["W"], row(p["mol_attend"]["b"]))

    heads_in_specs = [const2(s) for s in
                      [(B, D), (B, D),
                       (D, 128), (1, 128), (D, 512), (1, 512),
                       (512, 1024), (1, 1024), (1024, D), (1, D),
                       (128, 1312), (D, 1312), (1, 1312),
                       (1312, 656), (1, 656), (656, 1), (1, 1)]]
    mol_prediction = pl.pallas_call(
        _heads_body,
        grid=(1,),
        in_specs=heads_in_specs,
        out_specs=pl.BlockSpec((B, 1), lambda i: (0, 0)),
        out_shape=jax.ShapeDtypeStruct((B, 1), f32),
    )(mol_feature, descriptors.astype(f32),
      p["fc_g1"]["W"], row(p["fc_g1"]["b"]),
      p["sn1"]["W"], row(p["sn1"]["b"]),
      p["sn2"]["W"], row(p["sn2"]["b"]),
      p["sn3"]["W"], row(p["sn3"]["b"]),
      p["fc1"]["W"][:128, :], p["fc1"]["W"][128:, :], row(p["fc1"]["b"]),
      p["fc2"]["W"], row(p["fc2"]["b"]),
      p["out"]["W"], row(p["out"]["b"]))

    return (atom_feature, mol_prediction)


# speedup vs baseline: 13.2358x; 1.5597x over previous
import jax
import jax.numpy as jnp
from jax.experimental import pallas as pl

B = 256
L = 128
M = 6
D = 200
IN_FEAT = 39
BOND_FEAT = 10
RADIUS = 3
T_STEPS = 2
NB = 8
NEG = -900000000.0


def _dot(a, b):
    return jax.lax.dot_general(
        a, b, (((1,), (0,)), ((), ())),
        preferred_element_type=jnp.float32,
        precision=jax.lax.Precision.DEFAULT)


def _dot_exact(a, b):
    return jax.lax.dot_general(
        a, b, (((1,), (0,)), ((), ())),
        preferred_element_type=jnp.float32,
        precision=jax.lax.Precision.HIGHEST)


def _leaky(x):
    return jnp.where(x >= 0, x, 0.01 * x)


def _elu(x):
    return jnp.where(x > 0, x, jnp.exp(jnp.minimum(x, 0.0)) - 1.0)


def _gnn_body(atom_ref, bond_ref, aidx_ref, bidx_ref, mask_ref,
              aW, aB, nWa, nWb, nB_,
              g_ih, g_hh, g_bih, g_bhh,
              al_wA, al_wN_col, al_wN_row, al_b,
              at_W, at_b,
              mg_ih, mg_hh, mg_bih, mg_bhh,
              ma_wA, ma_wN, ma_b,
              mt_W, mt_b,
              out_af, out_mf):
    f32 = jnp.float32

    def gru(x, h, ih, hh, bih, bhh, base):
        r = jax.nn.sigmoid(_dot(x, ih[base + 0]) + bih[base + 0]
                           + _dot(h, hh[base + 0]) + bhh[base + 0])
        z = jax.nn.sigmoid(_dot(x, ih[base + 1]) + bih[base + 1]
                           + _dot(h, hh[base + 1]) + bhh[base + 1])
        n = jnp.tanh(_dot(x, ih[base + 2]) + bih[base + 2]
                     + r * (_dot(h, hh[base + 2]) + bhh[base + 2]))
        return (1.0 - z) * n + z * h

    x2 = atom_ref[...].reshape(NB * L, IN_FEAT)
    bd2 = bond_ref[...].reshape(NB * L, BOND_FEAT)
    af = _leaky(_dot(x2, aW[...]) + aB[...])
    pa = _dot(x2, nWa[...])
    pb = _dot(bd2, nWb[...]) + nB_[...]

    iota = jax.lax.broadcasted_iota(jnp.int32, (L, L), 1)

    def hilo(x):
        hi = x.astype(jnp.bfloat16).astype(f32)
        return hi, x - hi

    sA0 = _dot(af, al_wA[0]) + al_b[0]
    pa_hi, pa_lo = hilo(pa)
    pb_hi, pb_lo = hilo(pb)
    ctx_rows = []
    for mol in range(NB):
        sl = slice(mol * L, (mol + 1) * L)
        aidx_m = aidx_ref[mol]
        bidx_m = bidx_ref[mol]
        sA_m = sA0[sl, :]
        oh_as, oh_bs, pads = [], [], []
        for m in range(M):
            a_col = aidx_m[:, m:m + 1]
            b_col = bidx_m[:, m:m + 1]
            oh_as.append((a_col == iota).astype(f32))
            oh_bs.append((b_col == iota).astype(f32))
            pads.append(a_col == (L - 1))
        OHa = jnp.concatenate(oh_as, axis=0)
        OHb = jnp.concatenate(oh_bs, axis=0)
        ga = _dot(OHa, pa_hi[sl, :]) + _dot(OHa, pa_lo[sl, :])
        gb = _dot(OHb, pb_hi[sl, :]) + _dot(OHb, pb_lo[sl, :])
        nf = _leaky(ga + gb)
        s_nf = _dot(nf, al_wN_col[0])
        scores = [_leaky(sA_m + s_nf[m * L:(m + 1) * L, :])
                  + jnp.where(pads[m], NEG, 0.0) for m in range(M)]
        mx = scores[0]
        for m in range(1, M):
            mx = jnp.maximum(mx, scores[m])
        es = [jnp.exp(s - mx) for s in scores]
        ssum = es[0]
        for m in range(1, M):
            ssum = ssum + es[m]
        nbr_t = _dot(nf, at_W[0]) + at_b[0]
        ctx = jnp.zeros((L, D), f32)
        for m in range(M):
            w = jnp.where(pads[m], 0.0, es[m] / ssum)
            ctx = ctx + w * nbr_t[m * L:(m + 1) * L, :]
        ctx_rows.append(ctx)
    ctx_all = _elu(jnp.concatenate(ctx_rows, axis=0))
    h = gru(ctx_all, af, g_ih, g_hh, g_bih, g_bhh, 0)
    act = jax.nn.relu(h)

    for r in range(1, RADIUS):
        proj = _dot(act, at_W[r])
        sA = _dot(act, al_wA[r]) + al_b[r]
        sN = _dot(act, al_wN_col[r])
        ctx_rows = []
        for mol in range(NB):
            sl = slice(mol * L, (mol + 1) * L)
            proj_m = proj[sl, :]
            sA_m, sN_m = sA[sl, :], sN[sl, :]
            aidx_m = aidx_ref[mol]
            ohs, pads = [], []
            for m in range(M):
                a_col = aidx_m[:, m:m + 1]
                ohs.append((a_col == iota).astype(f32))
                pads.append(a_col == (L - 1))
            OH = jnp.concatenate(ohs, axis=0)
            s_g = _dot_exact(OH, sN_m)
            scores = [_leaky(sA_m + s_g[m * L:(m + 1) * L, :])
                      + jnp.where(pads[m], NEG, 0.0) for m in range(M)]
            mx = scores[0]
            for m in range(1, M):
                mx = jnp.maximum(mx, scores[m])
            es = [jnp.exp(s - mx) for s in scores]
            ssum = es[0]
            for m in range(1, M):
                ssum = ssum + es[m]
            A = jnp.zeros((L, L), f32)
            wsum = jnp.zeros((L, 1), f32)
            for m in range(M):
                w = jnp.where(pads[m], 0.0, es[m] / ssum)
                A = A + w * ohs[m]
                wsum = wsum + w
            ctx = _dot_exact(A, proj_m) + wsum * at_b[r]
            ctx_rows.append(ctx)
        ctx_all = _elu(jnp.concatenate(ctx_rows, axis=0))
        h = gru(ctx_all, h, g_ih, g_hh, g_bih, g_bhh, 3 * r)
        act = jax.nn.relu(h)

    mf_rows = []
    for mol in range(NB):
        sl = slice(mol * L, (mol + 1) * L)
        mask_m = mask_ref[mol]
        mf_rows.append(jnp.sum(act[sl, :] * mask_m, axis=0, keepdims=True))
    mf = jnp.concatenate(mf_rows, axis=0)
    act_t = _dot(act, mt_W[...]) + mt_b[...]
    sN_mol = _dot(act, ma_wN[...])
    am = jax.nn.relu(mf)
    for _t in range(T_STEPS):
        sA_t = _dot(am, ma_wA[...]) + ma_b[...]
        mc_rows = []
        for mol in range(NB):
            sl = slice(mol * L, (mol + 1) * L)
            mask_m = mask_ref[mol]
            score = _leaky(sA_t[mol:mol + 1, :] + sN_mol[sl, :]) \
                + jnp.where(mask_m == 0.0, NEG, 0.0)
            mx = jnp.max(score, axis=0, keepdims=True)
            e = jnp.exp(score - mx)
            w = e / jnp.sum(e, axis=0, keepdims=True) * mask_m
            mc_rows.append(jnp.sum(w * act_t[sl, :], axis=0, keepdims=True))
        mc = _elu(jnp.concatenate(mc_rows, axis=0))
        mf = gru(mc, mf, mg_ih, mg_hh, mg_bih, mg_bhh, 0)
        am = jax.nn.relu(mf)

    out_af[...] = h.reshape(NB, L, D)
    out_mf[...] = mf


def _heads_body(mf_ref, desc_ref,
                g1W, g1b, s1W, s1b, s2W, s2b, s3W, s3b,
                f1Wa, f1Wb, f1b, f2W, f2b, oW, ob,
                out_ref):
    x = _dot(mf_ref[...], g1W[...]) + g1b[...]
    d1 = jax.nn.relu(_dot(desc_ref[...], s1W[...]) + s1b[...])
    d2 = jax.nn.relu(_dot(d1, s2W[...]) + s2b[...])
    d3 = _dot(d2, s3W[...]) + s3b[...]
    m1 = jax.nn.relu(_dot(x, f1Wa[...]) + _dot(d3, f1Wb[...]) + f1b[...])
    m2 = jax.nn.relu(_dot(m1, f2W[...]) + f2b[...])
    out_ref[...] = _dot(m2, oW[...]) + ob[...]


def kernel(atom_list, bond_list, atom_degree_list, bond_degree_list,
           atom_mask, descriptors, params):
    p = params
    f32 = jnp.float32
    aidx = atom_degree_list.astype(jnp.int32)
    bidx = bond_degree_list.astype(jnp.int32)
    mask3 = atom_mask.astype(f32).reshape(B, L, 1)

    def row(b):
        return b.reshape(1, -1).astype(f32)

    def pack_gru(gs):
        ih = jnp.stack([g["w_ih"][i * D:(i + 1) * D, :].T
                        for g in gs for i in range(3)])
        hh = jnp.stack([g["w_hh"][i * D:(i + 1) * D, :].T
                        for g in gs for i in range(3)])
        bih = jnp.stack([g["b_ih"][i * D:(i + 1) * D].reshape(1, D)
                         for g in gs for i in range(3)])
        bhh = jnp.stack([g["b_hh"][i * D:(i + 1) * D].reshape(1, D)
                         for g in gs for i in range(3)])
        return ih, hh, bih, bhh

    g_ih, g_hh, g_bih, g_bhh = pack_gru(p["gru"])
    mg_ih, mg_hh, mg_bih, mg_bhh = pack_gru([p["mol_gru"]])

    al_wA = jnp.stack([a["W"][:D, :] for a in p["align"]])
    al_wN_col = jnp.stack([a["W"][D:, :] for a in p["align"]])
    al_wN_row = jnp.stack([a["W"][D:, :].T for a in p["align"]])
    al_b = jnp.stack([a["b"].reshape(1, 1) for a in p["align"]])
    at_W = jnp.stack([a["W"] for a in p["attend"]])
    at_b = jnp.stack([a["b"].reshape(1, D) for a in p["attend"]])

    nWa = p["nbr_lin"]["W"][:IN_FEAT, :]
    nWb = p["nbr_lin"]["W"][IN_FEAT:, :]

    grid = B // NB
    const2 = lambda shape: pl.BlockSpec(shape, lambda i: (0,) * len(shape))
    gnn_in_specs = [
        pl.BlockSpec((NB, L, IN_FEAT), lambda i: (i, 0, 0)),
        pl.BlockSpec((NB, L, BOND_FEAT), lambda i: (i, 0, 0)),
        pl.BlockSpec((NB, L, M), lambda i: (i, 0, 0)),
        pl.BlockSpec((NB, L, M), lambda i: (i, 0, 0)),
        pl.BlockSpec((NB, L, 1), lambda i: (i, 0, 0)),
        const2((IN_FEAT, D)), const2((1, D)),
        const2((IN_FEAT, D)), const2((BOND_FEAT, D)), const2((1, D)),
        const2((9, D, D)), const2((9, D, D)),
        const2((9, 1, D)), const2((9, 1, D)),
        const2((3, D, 1)), const2((3, D, 1)), const2((3, 1, D)),
        const2((3, 1, 1)),
        const2((3, D, D)), const2((3, 1, D)),
        const2((3, D, D)), const2((3, D, D)),
        const2((3, 1, D)), const2((3, 1, D)),
        const2((D, 1)), const2((D, 1)), const2((1, 1)),
        const2((D, D)), const2((1, D)),
    ]
    atom_feature, mol_feature = pl.pallas_call(
        _gnn_body,
        grid=(grid,),
        in_specs=gnn_in_specs,
        out_specs=[
            pl.BlockSpec((NB, L, D), lambda i: (i, 0, 0)),
            pl.BlockSpec((NB, D), lambda i: (i, 0)),
        ],
        out_shape=[
            jax.ShapeDtypeStruct((B, L, D), f32),
            jax.ShapeDtypeStruct((B, D), f32),
        ],
    )(atom_list.astype(f32), bond_list.astype(f32), aidx, bidx, mask3,
      p["atom_lin"]["W"], row(p["atom_lin"]["b"]),
      nWa, nWb, row(p["nbr_lin"]["b"]),
      g_ih, g_hh, g_bih, g_bhh,
      al_wA, al_wN_col, al_wN_row, al_b,
      at_W, at_b,
      mg_ih, mg_hh, mg_bih, mg_bhh,
      p["mol_align"]["W"][:D, :], p["mol_align"]["W"][D:, :],
      p["mol_align"]["b"].reshape(1, 1),
      p["mol_attend"]["W"], row(p["mol_attend"]["b"]))

    heads_in_specs = [const2(s) for s in
                      [(B, D), (B, D),
                       (D, 128), (1, 128), (D, 512), (1, 512),
                       (512, 1024), (1, 1024), (1024, D), (1, D),
                       (128, 1312), (D, 1312), (1, 1312),
                       (1312, 656), (1, 656), (656, 1), (1, 1)]]
    mol_prediction = pl.pallas_call(
        _heads_body,
        grid=(1,),
        in_specs=heads_in_specs,
        out_specs=pl.BlockSpec((B, 1), lambda i: (0, 0)),
        out_shape=jax.ShapeDtypeStruct((B, 1), f32),
    )(mol_feature, descriptors.astype(f32),
      p["fc_g1"]["W"], row(p["fc_g1"]["b"]),
      p["sn1"]["W"], row(p["sn1"]["b"]),
      p["sn2"]["W"], row(p["sn2"]["b"]),
      p["sn3"]["W"], row(p["sn3"]["b"]),
      p["fc1"]["W"][:128, :], p["fc1"]["W"][128:, :], row(p["fc1"]["b"]),
      p["fc2"]["W"], row(p["fc2"]["b"]),
      p["out"]["W"], row(p["out"]["b"]))

    return (atom_feature, mol_prediction)


# speedup vs baseline: 17.3797x; 1.3131x over previous
import jax
import jax.numpy as jnp
from jax.experimental import pallas as pl

B = 256
L = 128
M = 6
D = 200
IN_FEAT = 39
BOND_FEAT = 10
RADIUS = 3
T_STEPS = 2
NB = 8
NEG = -900000000.0


def _dot(a, b):
    return jax.lax.dot_general(
        a, b, (((1,), (0,)), ((), ())),
        preferred_element_type=jnp.float32,
        precision=jax.lax.Precision.DEFAULT)


def _leaky(x):
    return jnp.where(x >= 0, x, 0.01 * x)


def _elu(x):
    return jnp.where(x > 0, x, jnp.exp(jnp.minimum(x, 0.0)) - 1.0)


def _gnn_body(atom_ref, bond_ref, aidx_ref, bidx_ref, mask_ref,
              aW, aB, nWa, nWb, nB_,
              g_ih, g_hh, g_bih, g_bhh,
              al_wA, al_wN_col, al_b,
              at_W, at_b,
              mg_ih, mg_hh, mg_bih, mg_bhh,
              ma_wA, ma_wN, ma_b,
              mt_W, mt_b,
              out_af, out_mf):
    f32 = jnp.float32

    def gru(x, h, ih, hh, bih, bhh, base):
        r = jax.nn.sigmoid(_dot(x, ih[base + 0]) + bih[base + 0]
                           + _dot(h, hh[base + 0]) + bhh[base + 0])
        z = jax.nn.sigmoid(_dot(x, ih[base + 1]) + bih[base + 1]
                           + _dot(h, hh[base + 1]) + bhh[base + 1])
        n = jnp.tanh(_dot(x, ih[base + 2]) + bih[base + 2]
                     + r * (_dot(h, hh[base + 2]) + bhh[base + 2]))
        return (1.0 - z) * n + z * h

    x2 = atom_ref[...].reshape(NB * L, IN_FEAT)
    bd2 = bond_ref[...].reshape(NB * L, BOND_FEAT)
    af = _leaky(_dot(x2, aW[...]) + aB[...])
    pa = _dot(x2, nWa[...])
    pb = _dot(bd2, nWb[...]) + nB_[...]

    iota = jax.lax.broadcasted_iota(jnp.int32, (L, L), 1)

    def hilo(x):
        hi = x.astype(jnp.bfloat16).astype(f32)
        return hi, x - hi

    sA0 = _dot(af, al_wA[0]) + al_b[0]
    pa_hi, pa_lo = hilo(pa)
    pb_hi, pb_lo = hilo(pb)
    ctx_rows = []
    for mol in range(NB):
        sl = slice(mol * L, (mol + 1) * L)
        aidx_m = aidx_ref[mol]
        bidx_m = bidx_ref[mol]
        sA_m = sA0[sl, :]
        oh_as, oh_bs, pads = [], [], []
        for m in range(M):
            a_col = aidx_m[:, m:m + 1]
            b_col = bidx_m[:, m:m + 1]
            oh_as.append((a_col == iota).astype(f32))
            oh_bs.append((b_col == iota).astype(f32))
            pads.append(a_col == (L - 1))
        OHab = jnp.concatenate(
            [jnp.concatenate(oh_as, axis=0),
             jnp.concatenate(oh_bs, axis=0)], axis=1)
        p_hi = jnp.concatenate([pa_hi[sl, :], pb_hi[sl, :]], axis=0)
        p_lo = jnp.concatenate([pa_lo[sl, :], pb_lo[sl, :]], axis=0)
        nf = _leaky(_dot(OHab, p_hi) + _dot(OHab, p_lo))
        s_nf = _dot(nf, al_wN_col[0])
        scores = [_leaky(sA_m + s_nf[m * L:(m + 1) * L, :])
                  + jnp.where(pads[m], NEG, 0.0) for m in range(M)]
        mx = scores[0]
        for m in range(1, M):
            mx = jnp.maximum(mx, scores[m])
        es = [jnp.exp(s - mx) for s in scores]
        ssum = es[0]
        for m in range(1, M):
            ssum = ssum + es[m]
        nfb = nf.astype(jnp.bfloat16).astype(f32)
        S = jnp.zeros((L, D), f32)
        wsum = jnp.zeros((L, 1), f32)
        for m in range(M):
            w = jnp.where(pads[m], 0.0, es[m] / ssum)
            S = S + w * nfb[m * L:(m + 1) * L, :]
            wsum = wsum + w
        S_hi, S_lo = hilo(S)
        ctx = _dot(S_hi, at_W[0]) + _dot(S_lo, at_W[0]) + wsum * at_b[0]
        ctx_rows.append(ctx)
    ctx_all = _elu(jnp.concatenate(ctx_rows, axis=0))
    h = gru(ctx_all, af, g_ih, g_hh, g_bih, g_bhh, 0)
    act = jax.nn.relu(h)

    for r in range(1, RADIUS):
        proj = _dot(act, at_W[r])
        pj_hi, pj_lo = hilo(proj)
        sA = _dot(act, al_wA[r]) + al_b[r]
        sN = _dot(act, al_wN_col[r])
        sN_hi, sN_lo = hilo(sN)
        ctx_rows = []
        for mol in range(NB):
            sl = slice(mol * L, (mol + 1) * L)
            sA_m = sA[sl, :]
            aidx_m = aidx_ref[mol]
            ohs, pads = [], []
            for m in range(M):
                a_col = aidx_m[:, m:m + 1]
                ohs.append((a_col == iota).astype(f32))
                pads.append(a_col == (L - 1))
            OH = jnp.concatenate(ohs, axis=0)
            s_g = _dot(OH, sN_hi[sl, :]) + _dot(OH, sN_lo[sl, :])
            scores = [_leaky(sA_m + s_g[m * L:(m + 1) * L, :])
                      + jnp.where(pads[m], NEG, 0.0) for m in range(M)]
            mx = scores[0]
            for m in range(1, M):
                mx = jnp.maximum(mx, scores[m])
            es = [jnp.exp(s - mx) for s in scores]
            ssum = es[0]
            for m in range(1, M):
                ssum = ssum + es[m]
            A = jnp.zeros((L, L), f32)
            wsum = jnp.zeros((L, 1), f32)
            for m in range(M):
                w = jnp.where(pads[m], 0.0, es[m] / ssum)
                A = A + w * ohs[m]
                wsum = wsum + w
            A_hi, A_lo = hilo(A)
            ctx = (_dot(A_hi, pj_hi[sl, :]) + _dot(A_hi, pj_lo[sl, :])
                   + _dot(A_lo, pj_hi[sl, :]) + wsum * at_b[r])
            ctx_rows.append(ctx)
        ctx_all = _elu(jnp.concatenate(ctx_rows, axis=0))
        h = gru(ctx_all, h, g_ih, g_hh, g_bih, g_bhh, 3 * r)
        act = jax.nn.relu(h)

    mf_rows = []
    for mol in range(NB):
        sl = slice(mol * L, (mol + 1) * L)
        mask_m = mask_ref[mol]
        mf_rows.append(jnp.sum(act[sl, :] * mask_m, axis=0, keepdims=True))
    mf = jnp.concatenate(mf_rows, axis=0)
    act_t = _dot(act, mt_W[...]) + mt_b[...]
    sN_mol = _dot(act, ma_wN[...])
    am = jax.nn.relu(mf)
    for _t in range(T_STEPS):
        sA_t = _dot(am, ma_wA[...]) + ma_b[...]
        mc_rows = []
        for mol in range(NB):
            sl = slice(mol * L, (mol + 1) * L)
            mask_m = mask_ref[mol]
            score = _leaky(sA_t[mol:mol + 1, :] + sN_mol[sl, :]) \
                + jnp.where(mask_m == 0.0, NEG, 0.0)
            mx = jnp.max(score, axis=0, keepdims=True)
            e = jnp.exp(score - mx)
            w = e / jnp.sum(e, axis=0, keepdims=True) * mask_m
            mc_rows.append(jnp.sum(w * act_t[sl, :], axis=0, keepdims=True))
        mc = _elu(jnp.concatenate(mc_rows, axis=0))
        mf = gru(mc, mf, mg_ih, mg_hh, mg_bih, mg_bhh, 0)
        am = jax.nn.relu(mf)

    out_af[...] = h.reshape(NB, L, D)
    out_mf[...] = mf


def _heads_body(mf_ref, desc_ref,
                g1W, g1b, s1W, s1b, s2W, s2b, s3W, s3b,
                f1Wa, f1Wb, f1b, f2W, f2b, oW, ob,
                out_ref):
    x = _dot(mf_ref[...], g1W[...]) + g1b[...]
    d1 = jax.nn.relu(_dot(desc_ref[...], s1W[...]) + s1b[...])
    d2 = jax.nn.relu(_dot(d1, s2W[...]) + s2b[...])
    d3 = _dot(d2, s3W[...]) + s3b[...]
    m1 = jax.nn.relu(_dot(x, f1Wa[...]) + _dot(d3, f1Wb[...]) + f1b[...])
    m2 = jax.nn.relu(_dot(m1, f2W[...]) + f2b[...])
    out_ref[...] = _dot(m2, oW[...]) + ob[...]


def kernel(atom_list, bond_list, atom_degree_list, bond_degree_list,
           atom_mask, descriptors, params):
    p = params
    f32 = jnp.float32
    aidx = atom_degree_list.astype(jnp.int32)
    bidx = bond_degree_list.astype(jnp.int32)
    mask3 = atom_mask.astype(f32).reshape(B, L, 1)

    def row(b):
        return b.reshape(1, -1).astype(f32)

    def pack_gru(gs):
        ih = jnp.stack([g["w_ih"][i * D:(i + 1) * D, :].T
                        for g in gs for i in range(3)])
        hh = jnp.stack([g["w_hh"][i * D:(i + 1) * D, :].T
                        for g in gs for i in range(3)])
        bih = jnp.stack([g["b_ih"][i * D:(i + 1) * D].reshape(1, D)
                         for g in gs for i in range(3)])
        bhh = jnp.stack([g["b_hh"][i * D:(i + 1) * D].reshape(1, D)
                         for g in gs for i in range(3)])
        return ih, hh, bih, bhh

    g_ih, g_hh, g_bih, g_bhh = pack_gru(p["gru"])
    mg_ih, mg_hh, mg_bih, mg_bhh = pack_gru([p["mol_gru"]])

    al_wA = jnp.stack([a["W"][:D, :] for a in p["align"]])
    al_wN_col = jnp.stack([a["W"][D:, :] for a in p["align"]])
    al_b = jnp.stack([a["b"].reshape(1, 1) for a in p["align"]])
    at_W = jnp.stack([a["W"] for a in p["attend"]])
    at_b = jnp.stack([a["b"].reshape(1, D) for a in p["attend"]])

    nWa = p["nbr_lin"]["W"][:IN_FEAT, :]
    nWb = p["nbr_lin"]["W"][IN_FEAT:, :]

    grid = B // NB
    const2 = lambda shape: pl.BlockSpec(shape, lambda i: (0,) * len(shape))
    gnn_in_specs = [
        pl.BlockSpec((NB, L, IN_FEAT), lambda i: (i, 0, 0)),
        pl.BlockSpec((NB, L, BOND_FEAT), lambda i: (i, 0, 0)),
        pl.BlockSpec((NB, L, M), lambda i: (i, 0, 0)),
        pl.BlockSpec((NB, L, M), lambda i: (i, 0, 0)),
        pl.BlockSpec((NB, L, 1), lambda i: (i, 0, 0)),
        const2((IN_FEAT, D)), const2((1, D)),
        const2((IN_FEAT, D)), const2((BOND_FEAT, D)), const2((1, D)),
        const2((9, D, D)), const2((9, D, D)),
        const2((9, 1, D)), const2((9, 1, D)),
        const2((3, D, 1)), const2((3, D, 1)),
        const2((3, 1, 1)),
        const2((3, D, D)), const2((3, 1, D)),
        const2((3, D, D)), const2((3, D, D)),
        const2((3, 1, D)), const2((3, 1, D)),
        const2((D, 1)), const2((D, 1)), const2((1, 1)),
        const2((D, D)), const2((1, D)),
    ]
    atom_feature, mol_feature = pl.pallas_call(
        _gnn_body,
        grid=(grid,),
        in_specs=gnn_in_specs,
        out_specs=[
            pl.BlockSpec((NB, L, D), lambda i: (i, 0, 0)),
            pl.BlockSpec((NB, D), lambda i: (i, 0)),
        ],
        out_shape=[
            jax.ShapeDtypeStruct((B, L, D), f32),
            jax.ShapeDtypeStruct((B, D), f32),
        ],
    )(atom_list.astype(f32), bond_list.astype(f32), aidx, bidx, mask3,
      p["atom_lin"]["W"], row(p["atom_lin"]["b"]),
      nWa, nWb, row(p["nbr_lin"]["b"]),
      g_ih, g_hh, g_bih, g_bhh,
      al_wA, al_wN_col, al_b,
      at_W, at_b,
      mg_ih, mg_hh, mg_bih, mg_bhh,
      p["mol_align"]["W"][:D, :], p["mol_align"]["W"][D:, :],
      p["mol_align"]["b"].reshape(1, 1),
      p["mol_attend"]["W"], row(p["mol_attend"]["b"]))

    heads_in_specs = [const2(s) for s in
                      [(B, D), (B, D),
                       (D, 128), (1, 128), (D, 512), (1, 512),
                       (512, 1024), (1, 1024), (1024, D), (1, D),
                       (128, 1312), (D, 1312), (1, 1312),
                       (1312, 656), (1, 656), (656, 1), (1, 1)]]
    mol_prediction = pl.pallas_call(
        _heads_body,
        grid=(1,),
        in_specs=heads_in_specs,
        out_specs=pl.BlockSpec((B, 1), lambda i: (0, 0)),
        out_shape=jax.ShapeDtypeStruct((B, 1), f32),
    )(mol_feature, descriptors.astype(f32),
      p["fc_g1"]["W"], row(p["fc_g1"]["b"]),
      p["sn1"]["W"], row(p["sn1"]["b"]),
      p["sn2"]["W"], row(p["sn2"]["b"]),
      p["sn3"]["W"], row(p["sn3"]["b"]),
      p["fc1"]["W"][:128, :], p["fc1"]["W"][128:, :], row(p["fc1"]["b"]),
      p["fc2"]["W"], row(p["fc2"]["b"]),
      p["out"]["W"], row(p["out"]["b"]))

    return (atom_feature, mol_prediction)


# speedup vs baseline: 25.3787x; 1.4603x over previous
import jax
import jax.numpy as jnp
from jax.experimental import pallas as pl

B = 256
L = 128
M = 6
D = 200
IN_FEAT = 39
BOND_FEAT = 10
RADIUS = 3
T_STEPS = 2
NB = 8
NEG = -900000000.0


def _dot(a, b):
    return jax.lax.dot_general(
        a, b, (((1,), (0,)), ((), ())),
        preferred_element_type=jnp.float32,
        precision=jax.lax.Precision.DEFAULT)


def _dgT(a, b):
    return jax.lax.dot_general(
        a, b, (((0,), (1,)), ((), ())),
        preferred_element_type=jnp.float32,
        precision=jax.lax.Precision.DEFAULT)


def _dotT(a, b):
    return jax.lax.dot_general(
        a, b, (((0,), (0,)), ((), ())),
        preferred_element_type=jnp.float32,
        precision=jax.lax.Precision.DEFAULT)


def _leaky(x):
    return jnp.where(x >= 0, x, 0.01 * x)


def _elu(x):
    return jnp.where(x > 0, x, jnp.exp(jnp.minimum(x, 0.0)) - 1.0)


def _gnn_body(atom_ref, bond_ref, aidx_ref, bidx_ref, aidxT_ref, mask_ref,
              aW, aB, nWa, nWb, nB_,
              g_ih, g_hh, g_bih, g_bhh,
              al_wA, al_wN_col, al_b,
              at_W, at_b,
              mg_ih, mg_hh, mg_bih, mg_bhh,
              ma_wA, ma_wN, ma_b,
              mt_W, mt_b,
              out_af, out_mf):
    f32 = jnp.float32

    def gru(x, h, ih, hh, bih, bhh, base):
        r = jax.nn.sigmoid(_dot(x, ih[base + 0]) + bih[base + 0]
                           + _dot(h, hh[base + 0]) + bhh[base + 0])
        z = jax.nn.sigmoid(_dot(x, ih[base + 1]) + bih[base + 1]
                           + _dot(h, hh[base + 1]) + bhh[base + 1])
        n = jnp.tanh(_dot(x, ih[base + 2]) + bih[base + 2]
                     + r * (_dot(h, hh[base + 2]) + bhh[base + 2]))
        return (1.0 - z) * n + z * h

    x2 = atom_ref[...].reshape(NB * L, IN_FEAT)
    bd2 = bond_ref[...].reshape(NB * L, BOND_FEAT)
    af = _leaky(_dot(x2, aW[...]) + aB[...])
    pa = _dot(x2, nWa[...])
    pb = _dot(bd2, nWb[...]) + nB_[...]

    iota = jax.lax.broadcasted_iota(jnp.int32, (L, L), 1)
    iotaT = jax.lax.broadcasted_iota(jnp.int32, (L, L), 0)

    def hilo(x):
        hi = x.astype(jnp.bfloat16).astype(f32)
        return hi, x - hi

    def softmax_rows(scores_r, padTs):
        mx = scores_r[0]
        for m in range(1, M):
            mx = jnp.maximum(mx, scores_r[m])
        es = [jnp.exp(s - mx) for s in scores_r]
        ssum = es[0]
        for m in range(1, M):
            ssum = ssum + es[m]
        return [jnp.where(padTs[m], 0.0, es[m] / ssum) for m in range(M)]

    sA0 = _dgT(al_wA[0], af) + al_b[0]
    pa_hi, pa_lo = hilo(pa)
    pb_hi, pb_lo = hilo(pb)
    ctx_rows = []
    for mol in range(NB):
        sl = slice(mol * L, (mol + 1) * L)
        aidx_m = aidx_ref[mol]
        bidx_m = bidx_ref[mol]
        aidxT_m = aidxT_ref[mol]
        oh_as, oh_bs, padTs = [], [], []
        for m in range(M):
            oh_as.append((aidx_m[:, m:m + 1] == iota).astype(f32))
            oh_bs.append((bidx_m[:, m:m + 1] == iota).astype(f32))
            padTs.append(aidxT_m[m:m + 1, :] == (L - 1))
        OHab = jnp.concatenate(
            [jnp.concatenate(oh_as, axis=0),
             jnp.concatenate(oh_bs, axis=0)], axis=1)
        p_hi = jnp.concatenate([pa_hi[sl, :], pb_hi[sl, :]], axis=0)
        p_lo = jnp.concatenate([pa_lo[sl, :], pb_lo[sl, :]], axis=0)
        nf = _leaky(_dot(OHab, p_hi) + _dot(OHab, p_lo))
        s_nf = _dgT(al_wN_col[0], nf)
        sA_m = sA0[:, mol * L:(mol + 1) * L]
        scores_r = [_leaky(sA_m + s_nf[:, m * L:(m + 1) * L])
                    + jnp.where(padTs[m], NEG, 0.0) for m in range(M)]
        w_rows = softmax_rows(scores_r, padTs)
        Wc = jnp.transpose(jnp.concatenate(w_rows, axis=0))
        nfb = nf.astype(jnp.bfloat16).astype(f32)
        S = jnp.zeros((L, D), f32)
        for m in range(M):
            S = S + Wc[:, m:m + 1] * nfb[m * L:(m + 1) * L, :]
        wsum = jnp.sum(Wc, axis=1, keepdims=True)
        S_hi, S_lo = hilo(S)
        ctx = _dot(S_hi, at_W[0]) + _dot(S_lo, at_W[0]) + wsum * at_b[0]
        ctx_rows.append(ctx)
    ctx_all = _elu(jnp.concatenate(ctx_rows, axis=0))
    h = gru(ctx_all, af, g_ih, g_hh, g_bih, g_bhh, 0)
    act = jax.nn.relu(h)

    for r in range(1, RADIUS):
        pjb = _dot(act, at_W[r]) + at_b[r]
        pj_hi, pj_lo = hilo(pjb)
        sA = _dgT(al_wA[r], act) + al_b[r]
        sN = _dgT(al_wN_col[r], act)
        sN_hi, sN_lo = hilo(sN)
        ctx_rows = []
        for mol in range(NB):
            sl = slice(mol * L, (mol + 1) * L)
            c0 = mol * L
            aidxT_m = aidxT_ref[mol]
            ohTs, padTs = [], []
            for m in range(M):
                a_row = aidxT_m[m:m + 1, :]
                ohTs.append((a_row == iotaT).astype(f32))
                padTs.append(a_row == (L - 1))
            OHT = jnp.concatenate(ohTs, axis=1)
            s_g = (_dot(sN_hi[:, c0:c0 + L], OHT)
                   + _dot(sN_lo[:, c0:c0 + L], OHT))
            sA_m = sA[:, c0:c0 + L]
            scores_r = [_leaky(sA_m + s_g[:, m * L:(m + 1) * L])
                        + jnp.where(padTs[m], NEG, 0.0) for m in range(M)]
            w_rows = softmax_rows(scores_r, padTs)
            AT = jnp.zeros((L, L), f32)
            for m in range(M):
                AT = AT + w_rows[m] * ohTs[m]
            AT_hi, AT_lo = hilo(AT)
            ctx = (_dotT(AT_hi, pj_hi[sl, :]) + _dotT(AT_hi, pj_lo[sl, :])
                   + _dotT(AT_lo, pj_hi[sl, :]))
            ctx_rows.append(ctx)
        ctx_all = _elu(jnp.concatenate(ctx_rows, axis=0))
        h = gru(ctx_all, h, g_ih, g_hh, g_bih, g_bhh, 3 * r)
        act = jax.nn.relu(h)

    mf_rows = []
    for mol in range(NB):
        sl = slice(mol * L, (mol + 1) * L)
        mask_m = mask_ref[mol]
        mf_rows.append(jnp.sum(act[sl, :] * mask_m, axis=0, keepdims=True))
    mf = jnp.concatenate(mf_rows, axis=0)
    act_t = _dot(act, mt_W[...]) + mt_b[...]
    sN_mol = _dot(act, ma_wN[...])
    am = jax.nn.relu(mf)
    for _t in range(T_STEPS):
        sA_t = _dot(am, ma_wA[...]) + ma_b[...]
        mc_rows = []
        for mol in range(NB):
            sl = slice(mol * L, (mol + 1) * L)
            mask_m = mask_ref[mol]
            score = _leaky(sA_t[mol:mol + 1, :] + sN_mol[sl, :]) \
                + jnp.where(mask_m == 0.0, NEG, 0.0)
            mx = jnp.max(score, axis=0, keepdims=True)
            e = jnp.exp(score - mx)
            w = e / jnp.sum(e, axis=0, keepdims=True) * mask_m
            mc_rows.append(jnp.sum(w * act_t[sl, :], axis=0, keepdims=True))
        mc = _elu(jnp.concatenate(mc_rows, axis=0))
        mf = gru(mc, mf, mg_ih, mg_hh, mg_bih, mg_bhh, 0)
        am = jax.nn.relu(mf)

    out_af[...] = h.reshape(NB, L, D)
    out_mf[...] = mf


def _heads_body(mf_ref, desc_ref,
                g1W, g1b, s1W, s1b, s2W, s2b, s3W, s3b,
                f1Wa, f1Wb, f1b, f2W, f2b, oW, ob,
                out_ref):
    x = _dot(mf_ref[...], g1W[...]) + g1b[...]
    d1 = jax.nn.relu(_dot(desc_ref[...], s1W[...]) + s1b[...])
    d2 = jax.nn.relu(_dot(d1, s2W[...]) + s2b[...])
    d3 = _dot(d2, s3W[...]) + s3b[...]
    m1 = jax.nn.relu(_dot(x, f1Wa[...]) + _dot(d3, f1Wb[...]) + f1b[...])
    m2 = jax.nn.relu(_dot(m1, f2W[...]) + f2b[...])
    out_ref[...] = _dot(m2, oW[...]) + ob[...]


def kernel(atom_list, bond_list, atom_degree_list, bond_degree_list,
           atom_mask, descriptors, params):
    p = params
    f32 = jnp.float32
    aidx = atom_degree_list.astype(jnp.int32)
    bidx = bond_degree_list.astype(jnp.int32)
    mask3 = atom_mask.astype(f32).reshape(B, L, 1)

    def row(b):
        return b.reshape(1, -1).astype(f32)

    def pack_gru(gs):
        ih = jnp.stack([g["w_ih"][i * D:(i + 1) * D, :].T
                        for g in gs for i in range(3)])
        hh = jnp.stack([g["w_hh"][i * D:(i + 1) * D, :].T
                        for g in gs for i in range(3)])
        bih = jnp.stack([g["b_ih"][i * D:(i + 1) * D].reshape(1, D)
                         for g in gs for i in range(3)])
        bhh = jnp.stack([g["b_hh"][i * D:(i + 1) * D].reshape(1, D)
                         for g in gs for i in range(3)])
        return ih, hh, bih, bhh

    g_ih, g_hh, g_bih, g_bhh = pack_gru(p["gru"])
    mg_ih, mg_hh, mg_bih, mg_bhh = pack_gru([p["mol_gru"]])

    al_wA = jnp.stack([a["W"][:D, :] for a in p["align"]])
    al_wN_col = jnp.stack([a["W"][D:, :] for a in p["align"]])
    al_b = jnp.stack([a["b"].reshape(1, 1) for a in p["align"]])
    at_W = jnp.stack([a["W"] for a in p["attend"]])
    at_b = jnp.stack([a["b"].reshape(1, D) for a in p["attend"]])

    nWa = p["nbr_lin"]["W"][:IN_FEAT, :]
    nWb = p["nbr_lin"]["W"][IN_FEAT:, :]

    grid = B // NB
    const2 = lambda shape: pl.BlockSpec(shape, lambda i: (0,) * len(shape))
    gnn_in_specs = [
        pl.BlockSpec((NB, L, IN_FEAT), lambda i: (i, 0, 0)),
        pl.BlockSpec((NB, L, BOND_FEAT), lambda i: (i, 0, 0)),
        pl.BlockSpec((NB, L, M), lambda i: (i, 0, 0)),
        pl.BlockSpec((NB, L, M), lambda i: (i, 0, 0)),
        pl.BlockSpec((NB, M, L), lambda i: (i, 0, 0)),
        pl.BlockSpec((NB, L, 1), lambda i: (i, 0, 0)),
        const2((IN_FEAT, D)), const2((1, D)),
        const2((IN_FEAT, D)), const2((BOND_FEAT, D)), const2((1, D)),
        const2((9, D, D)), const2((9, D, D)),
        const2((9, 1, D)), const2((9, 1, D)),
        const2((3, D, 1)), const2((3, D, 1)),
        const2((3, 1, 1)),
        const2((3, D, D)), const2((3, 1, D)),
        const2((3, D, D)), const2((3, D, D)),
        const2((3, 1, D)), const2((3, 1, D)),
        const2((D, 1)), const2((D, 1)), const2((1, 1)),
        const2((D, D)), const2((1, D)),
    ]
    atom_feature, mol_feature = pl.pallas_call(
        _gnn_body,
        grid=(grid,),
        in_specs=gnn_in_specs,
        out_specs=[
            pl.BlockSpec((NB, L, D), lambda i: (i, 0, 0)),
            pl.BlockSpec((NB, D), lambda i: (i, 0)),
        ],
        out_shape=[
            jax.ShapeDtypeStruct((B, L, D), f32),
            jax.ShapeDtypeStruct((B, D), f32),
        ],
    )(atom_list.astype(f32), bond_list.astype(f32), aidx, bidx,
      aidx.swapaxes(1, 2), mask3,
      p["atom_lin"]["W"], row(p["atom_lin"]["b"]),
      nWa, nWb, row(p["nbr_lin"]["b"]),
      g_ih, g_hh, g_bih, g_bhh,
      al_wA, al_wN_col, al_b,
      at_W, at_b,
      mg_ih, mg_hh, mg_bih, mg_bhh,
      p["mol_align"]["W"][:D, :], p["mol_align"]["W"][D:, :],
      p["mol_align"]["b"].reshape(1, 1),
      p["mol_attend"]["W"], row(p["mol_attend"]["b"]))

    heads_in_specs = [const2(s) for s in
                      [(B, D), (B, D),
                       (D, 128), (1, 128), (D, 512), (1, 512),
                       (512, 1024), (1, 1024), (1024, D), (1, D),
                       (128, 1312), (D, 1312), (1, 1312),
                       (1312, 656), (1, 656), (656, 1), (1, 1)]]
    mol_prediction = pl.pallas_call(
        _heads_body,
        grid=(1,),
        in_specs=heads_in_specs,
        out_specs=pl.BlockSpec((B, 1), lambda i: (0, 0)),
        out_shape=jax.ShapeDtypeStruct((B, 1), f32),
    )(mol_feature, descriptors.astype(f32),
      p["fc_g1"]["W"], row(p["fc_g1"]["b"]),
      p["sn1"]["W"], row(p["sn1"]["b"]),
      p["sn2"]["W"], row(p["sn2"]["b"]),
      p["sn3"]["W"], row(p["sn3"]["b"]),
      p["fc1"]["W"][:128, :], p["fc1"]["W"][128:, :], row(p["fc1"]["b"]),
      p["fc2"]["W"], row(p["fc2"]["b"]),
      p["out"]["W"], row(p["out"]["b"]))

    return (atom_feature, mol_prediction)
